# baseline (speedup 1.0000x reference)
import functools
import math

import jax
import jax.numpy as jnp
from jax import lax
from jax.experimental import pallas as pl
from jax.experimental.pallas import tpu as pltpu
from jax.experimental.pallas import tpu_sc as plsc

_SP = 1.0
_SN = 64.0
_COS_M = math.cos(0.5)
_SIN_M = math.sin(0.5)

_NC = 2
_NS = 16
_NW = _NC * _NS

_B = 1024
_C = 100000
_CSPLIT = 99968
_ROWS_PER_W = _B // _NW
_CHUNK = 2944
_NFULL = 33
_TAIL = _CSPLIT - _NFULL * _CHUNK


def _sc_body(cos_hbm, out_hbm, in_v, out_v):
    wid = lax.axis_index("s") * _NC + lax.axis_index("c")
    r0 = wid * _ROWS_PER_W

    def band(row, width, c0):
        pltpu.sync_copy(cos_hbm.at[pl.ds(row, 8), pl.ds(c0, width)],
                        in_v.at[:, pl.ds(0, width)])

        def step(g, carry):
            base = pl.multiple_of(g * 16, 16)
            for r8 in range(8):
                x = in_v[r8, pl.ds(base, 16)]
                x = jnp.minimum(jnp.maximum(x, -1.0), 1.0)
                out_v[r8, pl.ds(base, 16)] = x * _SN
            return carry

        lax.fori_loop(0, width // 16, step, 0)
        pltpu.sync_copy(out_v.at[:, pl.ds(0, width)],
                        out_hbm.at[pl.ds(row, 8), pl.ds(c0, width)])

    for tr in range(_ROWS_PER_W // 8):
        row = r0 + tr * 8

        def chunk_step(ck, carry):
            c0 = pl.multiple_of(ck * _CHUNK, 128)
            band(row, _CHUNK, c0)
            return carry

        lax.fori_loop(0, _NFULL, chunk_step, 0)
        band(row, _TAIL, _NFULL * _CHUNK)


def _sc_stream(cosine):
    mesh = plsc.VectorSubcoreMesh(core_axis_name="c", subcore_axis_name="s")
    return pl.kernel(
        _sc_body,
        out_type=jax.ShapeDtypeStruct((_B, _C), jnp.float32),
        mesh=mesh,
        scratch_types=[
            pltpu.VMEM((8, _CHUNK), jnp.float32),
            pltpu.VMEM((8, _CHUNK), jnp.float32),
        ],
        compiler_params=pltpu.CompilerParams(use_tc_tiling_on_sc=True),
    )(cosine)


def _tc_strip_body(lab_ref, cos_ref, main_ref, out_ref):
    del main_ref
    x = jnp.clip(cos_ref[...], -1.0, 1.0)
    lab = lab_ref[...]
    col = _CSPLIT + jax.lax.broadcasted_iota(jnp.int32, x.shape, 1)
    is_t = col == lab
    t = jnp.max(jnp.where(is_t, x, -2.0), axis=1, keepdims=True)
    adj = (t * _COS_M - jnp.sqrt(jnp.maximum(1.0 - t * t, 0.0)) * _SIN_M) * _SP
    out_ref[...] = jnp.where(is_t, adj, x) * _SN


def _tc_strip(cosine, label, out_main):
    b, c = cosine.shape
    blk = _CSPLIT // 128
    return pl.pallas_call(
        _tc_strip_body,
        grid=(1,),
        in_specs=[
            pl.BlockSpec((b, 1), lambda i: (0, 0)),
            pl.BlockSpec((b, 128), lambda i: (0, blk)),
            pl.BlockSpec((b, 128), lambda i: (0, blk)),
        ],
        out_specs=pl.BlockSpec((b, 128), lambda i: (0, blk)),
        out_shape=jax.ShapeDtypeStruct((b, c), cosine.dtype),
        input_output_aliases={2: 0},
        compiler_params=pltpu.CompilerParams(
            dimension_semantics=("arbitrary",),
        ),
    )(label[:, None], cosine, out_main)


def kernel(cosine, label):
    out_main = _sc_stream(cosine)
    return _tc_strip(cosine, label, out_main)


# speedup vs baseline: 1.4407x; 1.4407x over previous
import functools
import math

import jax
import jax.numpy as jnp
from jax import lax
from jax.experimental import pallas as pl
from jax.experimental.pallas import tpu as pltpu
from jax.experimental.pallas import tpu_sc as plsc

_SP = 1.0
_SN = 64.0
_COS_M = math.cos(0.5)
_SIN_M = math.sin(0.5)

_NC = 2
_NS = 16
_NW = _NC * _NS

_B = 1024
_C = 100000
_CSPLIT = 99968
_ROWS_PER_W = _B // _NW
_NBAND = _ROWS_PER_W // 8
_CHUNK = 2944
_NFULL = 33
_TAIL = _CSPLIT - _NFULL * _CHUNK
_NK = _NBAND * _NFULL


def _sc_body(cos_hbm, lab_hbm, out_hbm, in_v, out_v, lab_v,
             s_in0, s_in1, s_out0, s_out1):
    s_in = (s_in0, s_in1)
    s_out = (s_out0, s_out1)
    wid = lax.axis_index("s") * _NC + lax.axis_index("c")
    r0 = wid * _ROWS_PER_W
    pltpu.sync_copy(lab_hbm.at[pl.ds(r0, _ROWS_PER_W)],
                    lab_v.at[pl.ds(0, _ROWS_PER_W)])

    lane = lax.iota(jnp.int32, 16)
    row_in_band = lane & 7

    def rc(k):
        band = k // _NFULL
        ck = k - band * _NFULL
        if isinstance(k, int):
            return band, r0 + band * 8, ck * _CHUNK
        return (band,
                pl.multiple_of(r0 + band * 8, 8),
                pl.multiple_of(ck * _CHUNK, 128))

    def start_in(k, p):
        _, row, c0 = rc(k)
        pltpu.make_async_copy(
            cos_hbm.at[pl.ds(row, 8), pl.ds(c0, _CHUNK)],
            in_v.at[p], s_in[p]).start()

    def wait_in(p):
        pltpu.make_async_copy(
            cos_hbm.at[pl.ds(0, 8), pl.ds(0, _CHUNK)],
            in_v.at[p], s_in[p]).wait()

    def start_out(k, p):
        _, row, c0 = rc(k)
        pltpu.make_async_copy(
            out_v.at[p],
            out_hbm.at[pl.ds(row, 8), pl.ds(c0, _CHUNK)], s_out[p]).start()

    def wait_out(p):
        pltpu.make_async_copy(
            out_v.at[p],
            out_hbm.at[pl.ds(0, 8), pl.ds(0, _CHUNK)], s_out[p]).wait()

    def compute(p, band, c0, width):
        in_ref = in_v.at[p]
        out_ref = out_v.at[p]

        @plsc.parallel_loop(0, width // 16, 1, unroll=2)
        def _loop(g):
            base = pl.multiple_of(g * 16, 16)
            for r8 in range(8):
                x = in_ref[r8, pl.ds(base, 16)]
                x = jnp.minimum(jnp.maximum(x, -1.0), 1.0)
                out_ref[r8, pl.ds(base, 16)] = x * _SN

        if isinstance(band, int):
            loff = band * 8
        else:
            loff = pl.multiple_of(band * 8, 8)
        lab16 = lab_v[pl.ds(loff, 16)]
        col16 = lab16 - c0
        valid = (lane < 8) & (col16 >= 0) & (col16 < width) & (lab16 >= 0)
        colp = jnp.where(valid, col16, 0)
        t = plsc.load_gather(in_ref, [row_in_band, colp], mask=valid)
        tcl = jnp.minimum(jnp.maximum(t, -1.0), 1.0)
        a = 1.0 - tcl * tcl
        bi = plsc.bitcast(a, jnp.int32)
        bi = 0x5F3759DF - lax.shift_right_logical(bi, 1)
        y = plsc.bitcast(bi, jnp.float32)
        y = y * (1.5 - 0.5 * a * y * y)
        y = y * (1.5 - 0.5 * a * y * y)
        y = y * (1.5 - 0.5 * a * y * y)
        s = a * y
        adj = (tcl * _COS_M - s * _SIN_M) * (_SP * _SN)
        plsc.store_scatter(out_ref, [row_in_band, colp], adj, mask=valid)

    start_in(0, 0)
    start_in(1, 1)
    for p in range(2):
        band, _, c0 = rc(p)
        wait_in(p)
        compute(p, band, c0, _CHUNK)
        start_out(p, p)
        start_in(p + 2, p)

    def body(i, carry):
        for p in range(2):
            k = i * 2 + p
            band, _, c0 = rc(k)
            wait_in(p)
            wait_out(p)
            compute(p, band, c0, _CHUNK)
            start_out(k, p)
            start_in(k + 2, p)
        return carry

    lax.fori_loop(1, (_NK - 2) // 2, body, 0)
    for p in range(2):
        k = _NK - 2 + p
        band, _, c0 = rc(k)
        wait_in(p)
        wait_out(p)
        compute(p, band, c0, _CHUNK)
        start_out(k, p)
    wait_out(0)
    wait_out(1)

    for band in range(_NBAND):
        row = r0 + band * 8
        c0 = _NFULL * _CHUNK
        pltpu.sync_copy(cos_hbm.at[pl.ds(row, 8), pl.ds(c0, _TAIL)],
                        in_v.at[0, :, pl.ds(0, _TAIL)])
        compute(0, band, c0, _TAIL)
        pltpu.sync_copy(out_v.at[0, :, pl.ds(0, _TAIL)],
                        out_hbm.at[pl.ds(row, 8), pl.ds(c0, _TAIL)])


def _sc_stream(cosine, label):
    mesh = plsc.VectorSubcoreMesh(core_axis_name="c", subcore_axis_name="s")
    return pl.kernel(
        _sc_body,
        out_type=jax.ShapeDtypeStruct((_B, _C), jnp.float32),
        mesh=mesh,
        scratch_types=[
            pltpu.VMEM((2, 8, _CHUNK), jnp.float32),
            pltpu.VMEM((2, 8, _CHUNK), jnp.float32),
            pltpu.VMEM((48,), jnp.int32),
            pltpu.SemaphoreType.DMA,
            pltpu.SemaphoreType.DMA,
            pltpu.SemaphoreType.DMA,
            pltpu.SemaphoreType.DMA,
        ],
        compiler_params=pltpu.CompilerParams(
            use_tc_tiling_on_sc=True, needs_layout_passes=False),
    )(cosine, label)


def _tc_strip_body(lab_ref, cos_ref, main_ref, out_ref):
    del main_ref
    x = jnp.clip(cos_ref[...], -1.0, 1.0)
    lab = lab_ref[...]
    col = _CSPLIT + jax.lax.broadcasted_iota(jnp.int32, x.shape, 1)
    is_t = col == lab
    t = jnp.max(jnp.where(is_t, x, -2.0), axis=1, keepdims=True)
    adj = (t * _COS_M - jnp.sqrt(jnp.maximum(1.0 - t * t, 0.0)) * _SIN_M) * _SP
    out_ref[...] = jnp.where(is_t, adj, x) * _SN


def _tc_strip(cosine, label, out_main):
    b, c = cosine.shape
    blk = _CSPLIT // 128
    return pl.pallas_call(
        _tc_strip_body,
        grid=(1,),
        in_specs=[
            pl.BlockSpec((b, 1), lambda i: (0, 0)),
            pl.BlockSpec((b, 128), lambda i: (0, blk)),
            pl.BlockSpec((b, 128), lambda i: (0, blk)),
        ],
        out_specs=pl.BlockSpec((b, 128), lambda i: (0, blk)),
        out_shape=jax.ShapeDtypeStruct((b, c), cosine.dtype),
        input_output_aliases={2: 0},
        compiler_params=pltpu.CompilerParams(
            dimension_semantics=("arbitrary",),
        ),
    )(label[:, None], cosine, out_main)


def kernel(cosine, label):
    out_main = _sc_stream(cosine, label)
    return _tc_strip(cosine, label, out_main)


# speedup vs baseline: 1.4448x; 1.0028x over previous
import functools
import math

import jax
import jax.numpy as jnp
from jax import lax
from jax.experimental import pallas as pl
from jax.experimental.pallas import tpu as pltpu
from jax.experimental.pallas import tpu_sc as plsc

_SP = 1.0
_SN = 64.0
_COS_M = math.cos(0.5)
_SIN_M = math.sin(0.5)

_NC = 2
_NS = 16
_NW = _NC * _NS

_B = 1024
_C = 100000
_CSPLIT = 99968
_ROWS_PER_W = _B // _NW
_NBAND = _ROWS_PER_W // 8
_CHUNK = 2560
_NFULL = 39
_TAIL = _CSPLIT - _NFULL * _CHUNK
_NK = _NBAND * _NFULL
_DEPTH = 3


def _sc_body(cos_hbm, lab_hbm, out_hbm, in_v, out_v, lab_v,
             s_in0, s_in1, s_in2, s_out0, s_out1, s_out2):
    s_in = (s_in0, s_in1, s_in2)
    s_out = (s_out0, s_out1, s_out2)
    wid = lax.axis_index("s") * _NC + lax.axis_index("c")
    r0 = wid * _ROWS_PER_W
    pltpu.sync_copy(lab_hbm.at[pl.ds(r0, _ROWS_PER_W)],
                    lab_v.at[pl.ds(0, _ROWS_PER_W)])

    lane = lax.iota(jnp.int32, 16)
    row_in_band = lane & 7

    def rc(k):
        band = k // _NFULL
        ck = k - band * _NFULL
        if isinstance(k, int):
            return band, r0 + band * 8, ck * _CHUNK
        return (band,
                pl.multiple_of(r0 + band * 8, 8),
                pl.multiple_of(ck * _CHUNK, 128))

    def start_in(k, p):
        _, row, c0 = rc(k)
        pltpu.make_async_copy(
            cos_hbm.at[pl.ds(row, 8), pl.ds(c0, _CHUNK)],
            in_v.at[p], s_in[p]).start()

    def wait_in(p):
        pltpu.make_async_copy(
            cos_hbm.at[pl.ds(0, 8), pl.ds(0, _CHUNK)],
            in_v.at[p], s_in[p]).wait()

    def start_out(k, p):
        _, row, c0 = rc(k)
        pltpu.make_async_copy(
            out_v.at[p],
            out_hbm.at[pl.ds(row, 8), pl.ds(c0, _CHUNK)], s_out[p]).start()

    def wait_out(p):
        pltpu.make_async_copy(
            out_v.at[p],
            out_hbm.at[pl.ds(0, 8), pl.ds(0, _CHUNK)], s_out[p]).wait()

    def compute(p, band, c0, width):
        in_ref = in_v.at[p]
        out_ref = out_v.at[p]

        @plsc.parallel_loop(0, width // 16, 1, unroll=2)
        def _loop(g):
            base = pl.multiple_of(g * 16, 16)
            for r8 in range(8):
                x = in_ref[r8, pl.ds(base, 16)]
                x = jnp.minimum(jnp.maximum(x, -1.0), 1.0)
                out_ref[r8, pl.ds(base, 16)] = x * _SN

        if isinstance(band, int):
            loff = band * 8
        else:
            loff = pl.multiple_of(band * 8, 8)
        lab16 = lab_v[pl.ds(loff, 16)]
        col16 = lab16 - c0
        valid = (lane < 8) & (col16 >= 0) & (col16 < width) & (lab16 >= 0)
        colp = jnp.where(valid, col16, 0)
        t = plsc.load_gather(in_ref, [row_in_band, colp], mask=valid)
        tcl = jnp.minimum(jnp.maximum(t, -1.0), 1.0)
        a = 1.0 - tcl * tcl
        bi = plsc.bitcast(a, jnp.int32)
        bi = 0x5F3759DF - lax.shift_right_logical(bi, 1)
        y = plsc.bitcast(bi, jnp.float32)
        y = y * (1.5 - 0.5 * a * y * y)
        y = y * (1.5 - 0.5 * a * y * y)
        y = y * (1.5 - 0.5 * a * y * y)
        s = a * y
        adj = (tcl * _COS_M - s * _SIN_M) * (_SP * _SN)
        plsc.store_scatter(out_ref, [row_in_band, colp], adj, mask=valid)

    for p in range(_DEPTH):
        start_in(p, p)
    for p in range(_DEPTH):
        band, _, c0 = rc(p)
        wait_in(p)
        compute(p, band, c0, _CHUNK)
        start_out(p, p)
        start_in(p + _DEPTH, p)

    def body(i, carry):
        for p in range(_DEPTH):
            k = i * _DEPTH + p
            band, _, c0 = rc(k)
            wait_in(p)
            wait_out(p)
            compute(p, band, c0, _CHUNK)
            start_out(k, p)
            start_in(k + _DEPTH, p)
        return carry

    lax.fori_loop(1, _NK // _DEPTH - 1, body, 0)
    for p in range(_DEPTH):
        k = _NK - _DEPTH + p
        band, _, c0 = rc(k)
        wait_in(p)
        wait_out(p)
        compute(p, band, c0, _CHUNK)
        start_out(k, p)
    for p in range(_DEPTH):
        wait_out(p)

    for band in range(_NBAND):
        row = r0 + band * 8
        c0 = _NFULL * _CHUNK
        pltpu.sync_copy(cos_hbm.at[pl.ds(row, 8), pl.ds(c0, _TAIL)],
                        in_v.at[0, :, pl.ds(0, _TAIL)])
        compute(0, band, c0, _TAIL)
        pltpu.sync_copy(out_v.at[0, :, pl.ds(0, _TAIL)],
                        out_hbm.at[pl.ds(row, 8), pl.ds(c0, _TAIL)])


def _sc_stream(cosine, label):
    mesh = plsc.VectorSubcoreMesh(core_axis_name="c", subcore_axis_name="s")
    return pl.kernel(
        _sc_body,
        out_type=jax.ShapeDtypeStruct((_B, _C), jnp.float32),
        mesh=mesh,
        scratch_types=[
            pltpu.VMEM((_DEPTH, 8, _CHUNK), jnp.float32),
            pltpu.VMEM((_DEPTH, 8, _CHUNK), jnp.float32),
            pltpu.VMEM((48,), jnp.int32),
            pltpu.SemaphoreType.DMA,
            pltpu.SemaphoreType.DMA,
            pltpu.SemaphoreType.DMA,
            pltpu.SemaphoreType.DMA,
            pltpu.SemaphoreType.DMA,
            pltpu.SemaphoreType.DMA,
        ],
        compiler_params=pltpu.CompilerParams(
            use_tc_tiling_on_sc=True, needs_layout_passes=False),
    )(cosine, label)


def _tc_strip_body(lab_ref, cos_ref, main_ref, out_ref):
    del main_ref
    x = jnp.clip(cos_ref[...], -1.0, 1.0)
    lab = lab_ref[...]
    col = _CSPLIT + jax.lax.broadcasted_iota(jnp.int32, x.shape, 1)
    is_t = col == lab
    t = jnp.max(jnp.where(is_t, x, -2.0), axis=1, keepdims=True)
    adj = (t * _COS_M - jnp.sqrt(jnp.maximum(1.0 - t * t, 0.0)) * _SIN_M) * _SP
    out_ref[...] = jnp.where(is_t, adj, x) * _SN


def _tc_strip(cosine, label, out_main):
    b, c = cosine.shape
    blk = _CSPLIT // 128
    return pl.pallas_call(
        _tc_strip_body,
        grid=(1,),
        in_specs=[
            pl.BlockSpec((b, 1), lambda i: (0, 0)),
            pl.BlockSpec((b, 128), lambda i: (0, blk)),
            pl.BlockSpec((b, 128), lambda i: (0, blk)),
        ],
        out_specs=pl.BlockSpec((b, 128), lambda i: (0, blk)),
        out_shape=jax.ShapeDtypeStruct((b, c), cosine.dtype),
        input_output_aliases={2: 0},
        compiler_params=pltpu.CompilerParams(
            dimension_semantics=("arbitrary",),
        ),
    )(label[:, None], cosine, out_main)


def kernel(cosine, label):
    out_main = _sc_stream(cosine, label)
    return _tc_strip(cosine, label, out_main)


# speedup vs baseline: 1.4462x; 1.0010x over previous
import math

import jax
import jax.numpy as jnp
from jax import lax
from jax.experimental import pallas as pl
from jax.experimental.pallas import tpu as pltpu
from jax.experimental.pallas import tpu_sc as plsc

_SP = 1.0
_SN = 64.0
_COS_M = math.cos(0.5)
_SIN_M = math.sin(0.5)

_NC = 2
_NS = 16
_NW = _NC * _NS

_B = 1024
_C = 100000
_CSPLIT = 99968
_ROWS_PER_W = _B // _NW
_NBAND = _ROWS_PER_W // 8
_CHUNK = 2560
_NFULL = 39
_TAIL = _CSPLIT - _NFULL * _CHUNK
_NK = _NBAND * _NFULL
_DEPTH = 3


def _sc_body(cos_hbm, lab_hbm, out_hbm, in_v, out_v, lab_v,
             s_in0, s_in1, s_in2, s_out0, s_out1, s_out2):
    s_in = (s_in0, s_in1, s_in2)
    s_out = (s_out0, s_out1, s_out2)
    wid = lax.axis_index("s") * _NC + lax.axis_index("c")
    r0 = wid * _ROWS_PER_W
    pltpu.sync_copy(lab_hbm.at[pl.ds(r0, _ROWS_PER_W)],
                    lab_v.at[pl.ds(0, _ROWS_PER_W)])

    lane = lax.iota(jnp.int32, 16)
    row_in_band = lane & 7

    def rc(k):
        band = k // _NFULL
        ck = k - band * _NFULL
        if isinstance(k, int):
            return band, r0 + band * 8, ck * _CHUNK
        return (band,
                pl.multiple_of(r0 + band * 8, 8),
                pl.multiple_of(ck * _CHUNK, 128))

    def start_in(k, p):
        _, row, c0 = rc(k)
        pltpu.make_async_copy(
            cos_hbm.at[pl.ds(row, 8), pl.ds(c0, _CHUNK)],
            in_v.at[p], s_in[p]).start()

    def wait_in(p):
        pltpu.make_async_copy(
            cos_hbm.at[pl.ds(0, 8), pl.ds(0, _CHUNK)],
            in_v.at[p], s_in[p]).wait()

    def start_out(k, p):
        _, row, c0 = rc(k)
        pltpu.make_async_copy(
            out_v.at[p],
            out_hbm.at[pl.ds(row, 8), pl.ds(c0, _CHUNK)], s_out[p]).start()

    def wait_out(p):
        pltpu.make_async_copy(
            out_v.at[p],
            out_hbm.at[pl.ds(0, 8), pl.ds(0, _CHUNK)], s_out[p]).wait()

    def compute(p, band, c0, width):
        in_ref = in_v.at[p]
        out_ref = out_v.at[p]

        @plsc.parallel_loop(0, width // 16, 1, unroll=2)
        def _loop(g):
            base = pl.multiple_of(g * 16, 16)
            for r8 in range(8):
                x = in_ref[r8, pl.ds(base, 16)]
                x = jnp.minimum(jnp.maximum(x, -1.0), 1.0)
                out_ref[r8, pl.ds(base, 16)] = x * _SN

        if isinstance(band, int):
            loff = band * 8
        else:
            loff = pl.multiple_of(band * 8, 8)
        lab16 = lab_v[pl.ds(loff, 16)]
        col16 = lab16 - c0
        valid = (lane < 8) & (col16 >= 0) & (col16 < width) & (lab16 >= 0)
        colp = jnp.where(valid, col16, 0)
        t = plsc.load_gather(in_ref, [row_in_band, colp], mask=valid)
        tcl = jnp.minimum(jnp.maximum(t, -1.0), 1.0)
        a = 1.0 - tcl * tcl
        bi = plsc.bitcast(a, jnp.int32)
        bi = 0x5F3759DF - lax.shift_right_logical(bi, 1)
        y = plsc.bitcast(bi, jnp.float32)
        y = y * (1.5 - 0.5 * a * y * y)
        y = y * (1.5 - 0.5 * a * y * y)
        y = y * (1.5 - 0.5 * a * y * y)
        s = a * y
        adj = (tcl * _COS_M - s * _SIN_M) * (_SP * _SN)
        plsc.store_scatter(out_ref, [row_in_band, colp], adj, mask=valid)

    for p in range(_DEPTH):
        start_in(p, p)
    for p in range(_DEPTH):
        band, _, c0 = rc(p)
        wait_in(p)
        compute(p, band, c0, _CHUNK)
        start_out(p, p)
        start_in(p + _DEPTH, p)

    def body(i, carry):
        for p in range(_DEPTH):
            k = i * _DEPTH + p
            band, _, c0 = rc(k)
            wait_in(p)
            wait_out(p)
            compute(p, band, c0, _CHUNK)
            start_out(k, p)
            start_in(k + _DEPTH, p)
        return carry

    lax.fori_loop(1, _NK // _DEPTH - 1, body, 0)
    for p in range(_DEPTH):
        k = _NK - _DEPTH + p
        band, _, c0 = rc(k)
        wait_in(p)
        wait_out(p)
        compute(p, band, c0, _CHUNK)
        start_out(k, p)
    for p in range(_DEPTH):
        wait_out(p)

    for band in range(_NBAND):
        row = r0 + band * 8
        c0 = _NFULL * _CHUNK
        pltpu.sync_copy(cos_hbm.at[pl.ds(row, 8), pl.ds(c0, _TAIL)],
                        in_v.at[0, :, pl.ds(0, _TAIL)])
        compute(0, band, c0, _TAIL)
        pltpu.sync_copy(out_v.at[0, :, pl.ds(0, _TAIL)],
                        out_hbm.at[pl.ds(row, 8), pl.ds(c0, _TAIL)])


def _sc_stream(cosine, label):
    mesh = plsc.VectorSubcoreMesh(core_axis_name="c", subcore_axis_name="s")
    return pl.kernel(
        _sc_body,
        out_type=jax.ShapeDtypeStruct((_B, _C), jnp.float32),
        mesh=mesh,
        scratch_types=[
            pltpu.VMEM((_DEPTH, 8, _CHUNK), jnp.float32),
            pltpu.VMEM((_DEPTH, 8, _CHUNK), jnp.float32),
            pltpu.VMEM((48,), jnp.int32),
            pltpu.SemaphoreType.DMA,
            pltpu.SemaphoreType.DMA,
            pltpu.SemaphoreType.DMA,
            pltpu.SemaphoreType.DMA,
            pltpu.SemaphoreType.DMA,
            pltpu.SemaphoreType.DMA,
        ],
        compiler_params=pltpu.CompilerParams(
            use_tc_tiling_on_sc=True, needs_layout_passes=False),
    )(cosine, label)


def _tc_strip_body(lab_ref, cos_ref, main_ref, out_ref):
    del main_ref
    x = jnp.clip(cos_ref[...], -1.0, 1.0)
    lab = lab_ref[...]
    col = _CSPLIT + jax.lax.broadcasted_iota(jnp.int32, x.shape, 1)
    is_t = col == lab
    t = jnp.max(jnp.where(is_t, x, -2.0), axis=1, keepdims=True)
    adj = (t * _COS_M - jnp.sqrt(jnp.maximum(1.0 - t * t, 0.0)) * _SIN_M) * _SP
    out_ref[...] = jnp.where(is_t, adj, x) * _SN


def _tc_strip(cosine, label, out_main):
    b, c = cosine.shape
    blk = _CSPLIT // 128
    return pl.pallas_call(
        _tc_strip_body,
        grid=(1,),
        in_specs=[
            pl.BlockSpec((b, 1), lambda i: (0, 0)),
            pl.BlockSpec((b, 128), lambda i: (0, blk)),
            pl.BlockSpec((b, 128), lambda i: (0, blk)),
        ],
        out_specs=pl.BlockSpec((b, 128), lambda i: (0, blk)),
        out_shape=jax.ShapeDtypeStruct((b, c), cosine.dtype),
        input_output_aliases={2: 0},
        compiler_params=pltpu.CompilerParams(
            dimension_semantics=("arbitrary",),
        ),
    )(label[:, None], cosine, out_main)


def kernel(cosine, label):
    out_main = _sc_stream(cosine, label)
    return _tc_strip(cosine, label, out_main)
